# Initial kernel scaffold; baseline (speedup 1.0000x reference)
import functools

import jax
import jax.numpy as jnp
from jax import lax
from jax.experimental import pallas as pl
from jax.experimental.pallas import tpu as pltpu
from jax.experimental.pallas import tpu_sc as plsc

_N = 8192
_K = 8192
_D = 256
_CW = 0.25

_PREP_BLK = 1024
_BT = 256


def _prep_body(c_ref, out_ref):
    c = c_ref[...]
    n = jnp.sqrt(jnp.sum(c * c, axis=1, keepdims=True))
    cn = c / jnp.maximum(n, 1e-12)
    out_ref[...] = cn.T


def _prep(codebook):
    return pl.pallas_call(
        _prep_body,
        grid=(_K // _PREP_BLK,),
        in_specs=[pl.BlockSpec((_PREP_BLK, _D), lambda i: (i, 0))],
        out_specs=pl.BlockSpec((_D, _PREP_BLK), lambda i: (0, i)),
        out_shape=jax.ShapeDtypeStruct((_D, _K), jnp.float32),
    )(codebook)


def _argmax_body(z_ref, cnt_ref, idx_ref):
    z = z_ref[...]
    zn = z / jnp.maximum(jnp.sqrt(jnp.sum(z * z, axis=1, keepdims=True)), 1e-12)
    sims = lax.dot_general(
        zn, cnt_ref[...], (((1,), (0,)), ((), ())),
        preferred_element_type=jnp.float32,
        precision=lax.Precision.HIGHEST,
    )
    m = jnp.max(sims, axis=1, keepdims=True)
    iota = lax.broadcasted_iota(jnp.int32, sims.shape, 1)
    idx = jnp.min(jnp.where(sims == m, iota, _K), axis=1, keepdims=True)
    idx_ref[...] = idx


def _argmax(z, cnt):
    return pl.pallas_call(
        _argmax_body,
        grid=(_N // _BT,),
        in_specs=[
            pl.BlockSpec((_BT, _D), lambda i: (i, 0)),
            pl.BlockSpec((_D, _K), lambda i: (0, 0)),
        ],
        out_specs=pl.BlockSpec((_BT, 1), lambda i: (i, 0)),
        out_shape=jax.ShapeDtypeStruct((_N, 1), jnp.int32),
    )(z, cnt)


def _gather_sc(codebook, idx):
    nw = 32
    bpw = _N // nw
    mesh = plsc.VectorSubcoreMesh(core_axis_name="c", subcore_axis_name="s")

    @functools.partial(
        pl.kernel, mesh=mesh,
        out_type=jax.ShapeDtypeStruct((_N, _D), jnp.float32),
        scratch_types=[
            pltpu.VMEM((bpw,), jnp.int32),
            pltpu.VMEM((bpw, _D), jnp.float32),
            pltpu.SemaphoreType.DMA,
        ],
    )
    def k(table_hbm, idx_hbm, out_hbm, idx_v, rows_v, sem):
        wid = lax.axis_index("s") * 2 + lax.axis_index("c")
        base = wid * bpw
        pltpu.sync_copy(idx_hbm.at[pl.ds(base, bpw)], idx_v)
        pltpu.async_copy(table_hbm.at[idx_v], rows_v, sem).wait()
        pltpu.sync_copy(rows_v, out_hbm.at[pl.ds(base, bpw)])

    return k(codebook, idx)


def _loss_body(z_ref, zq_ref, out_ref):
    i = pl.program_id(0)
    d = zq_ref[...] - z_ref[...]
    s = jnp.sum(d * d)

    @pl.when(i == 0)
    def _():
        out_ref[0, 0] = s

    @pl.when(i > 0)
    def _():
        out_ref[0, 0] += s


def _loss(z, zq):
    blk = 1024
    return pl.pallas_call(
        _loss_body,
        grid=(_N // blk,),
        in_specs=[
            pl.BlockSpec((blk, _D), lambda i: (i, 0)),
            pl.BlockSpec((blk, _D), lambda i: (i, 0)),
        ],
        out_specs=pl.BlockSpec((1, 1), lambda i: (0, 0)),
        out_shape=jax.ShapeDtypeStruct((1, 1), jnp.float32),
    )(z, zq)


def kernel(z, codebook):
    cnt = _prep(codebook)
    idx = _argmax(z, cnt).reshape(_N)
    zq = _gather_sc(codebook, idx)
    ssq = _loss(z, zq)
    loss = (_CW / (_N * _D)) * ssq[0, 0]
    return zq, loss


# speedup vs baseline: 1.0189x; 1.0189x over previous
import functools

import jax
import jax.numpy as jnp
from jax import lax
from jax.experimental import pallas as pl
from jax.experimental.pallas import tpu as pltpu
from jax.experimental.pallas import tpu_sc as plsc

_N = 8192
_K = 8192
_D = 256
_CW = 0.25

_PREP_BLK = 1024
_BT = 256
_APPROX = True


def _prep_body(c_ref, out_ref, cbr_ref):
    c = c_ref[...]
    n = jnp.sqrt(jnp.sum(c * c, axis=1, keepdims=True))
    cn = c * pl.reciprocal(jnp.maximum(n, 1e-12), approx=_APPROX)
    out_ref[...] = cn.astype(jnp.bfloat16).T
    cbr_ref[...] = c.astype(jnp.bfloat16).astype(jnp.float32)


def _prep(codebook):
    return pl.pallas_call(
        _prep_body,
        grid=(_K // _PREP_BLK,),
        in_specs=[pl.BlockSpec((_PREP_BLK, _D), lambda i: (i, 0))],
        out_specs=[
            pl.BlockSpec((_D, _PREP_BLK), lambda i: (0, i)),
            pl.BlockSpec((_PREP_BLK, _D), lambda i: (i, 0)),
        ],
        out_shape=[
            jax.ShapeDtypeStruct((_D, _K), jnp.bfloat16),
            jax.ShapeDtypeStruct((_K, _D), jnp.float32),
        ],
    )(codebook)


def _argmax_body(z_ref, cnt_ref, idx_ref):
    z = z_ref[...]
    n = jnp.sqrt(jnp.sum(z * z, axis=1, keepdims=True))
    zn = (z * pl.reciprocal(jnp.maximum(n, 1e-12), approx=_APPROX)
          ).astype(jnp.bfloat16)
    sims = lax.dot_general(
        zn, cnt_ref[...], (((1,), (0,)), ((), ())),
        preferred_element_type=jnp.float32,
        precision=lax.Precision.DEFAULT,
    )
    m_bf = sims.astype(jnp.bfloat16).astype(jnp.float32)
    mstar = jnp.max(m_bf, axis=1, keepdims=True)
    iota = lax.broadcasted_iota(jnp.int32, sims.shape, 1)
    lastex = jnp.max(jnp.where(sims > mstar, iota, -1), axis=1, keepdims=True)
    firstt = jnp.min(jnp.where(m_bf == mstar, iota, _K), axis=1, keepdims=True)
    idx_ref[...] = jnp.where(lastex >= 0, lastex, firstt)


def _argmax(z, cnt):
    return pl.pallas_call(
        _argmax_body,
        grid=(_N // _BT,),
        in_specs=[
            pl.BlockSpec((_BT, _D), lambda i: (i, 0)),
            pl.BlockSpec((_D, _K), lambda i: (0, 0)),
        ],
        out_specs=pl.BlockSpec((_BT, 1), lambda i: (i, 0)),
        out_shape=jax.ShapeDtypeStruct((_N, 1), jnp.int32),
    )(z, cnt)


def _gather_sc(table, idx):
    nw = 32
    bpw = _N // nw
    mesh = plsc.VectorSubcoreMesh(core_axis_name="c", subcore_axis_name="s")

    @functools.partial(
        pl.kernel, mesh=mesh,
        out_type=jax.ShapeDtypeStruct((_N, _D), jnp.float32),
        scratch_types=[
            pltpu.VMEM((bpw,), jnp.int32),
            pltpu.VMEM((bpw, _D), jnp.float32),
            pltpu.SemaphoreType.DMA,
        ],
    )
    def k(table_hbm, idx_hbm, out_hbm, idx_v, rows_v, sem):
        wid = lax.axis_index("s") * 2 + lax.axis_index("c")
        base = wid * bpw
        pltpu.sync_copy(idx_hbm.at[pl.ds(base, bpw)], idx_v)
        pltpu.async_copy(table_hbm.at[idx_v], rows_v, sem).wait()
        pltpu.sync_copy(rows_v, out_hbm.at[pl.ds(base, bpw)])

    return k(table, idx)


def _loss_body(z_ref, zq_ref, out_ref):
    i = pl.program_id(0)
    d = zq_ref[...] - z_ref[...]
    s = jnp.sum(d * d, axis=(0, 1), keepdims=True)

    @pl.when(i == 0)
    def _():
        out_ref[...] = s

    @pl.when(i > 0)
    def _():
        out_ref[...] += s


def _loss(z, zq):
    blk = 1024
    return pl.pallas_call(
        _loss_body,
        grid=(_N // blk,),
        in_specs=[
            pl.BlockSpec((blk, _D), lambda i: (i, 0)),
            pl.BlockSpec((blk, _D), lambda i: (i, 0)),
        ],
        out_specs=pl.BlockSpec((1, 1), lambda i: (0, 0)),
        out_shape=jax.ShapeDtypeStruct((1, 1), jnp.float32),
    )(z, zq)


def kernel(z, codebook):
    cnt, cbr = _prep(codebook)
    idx = _argmax(z, cnt).reshape(_N)
    zq = _gather_sc(cbr, idx)
    ssq = _loss(z, zq)
    loss = (_CW / (_N * _D)) * ssq[0, 0]
    return zq, loss


# speedup vs baseline: 1.3548x; 1.3297x over previous
import functools

import jax
import jax.numpy as jnp
from jax import lax
from jax.experimental import pallas as pl
from jax.experimental.pallas import tpu as pltpu
from jax.experimental.pallas import tpu_sc as plsc

_N = 8192
_K = 8192
_D = 256
_CW = 0.25

_PREP_BLK = 1024
_BT = 256
_APPROX = True


def _prep_body(c_ref, out_ref, cbr_ref):
    c = c_ref[...]
    n = jnp.sqrt(jnp.sum(c * c, axis=1, keepdims=True))
    cn = c * pl.reciprocal(jnp.maximum(n, 1e-12), approx=_APPROX)
    out_ref[...] = cn.astype(jnp.bfloat16).T
    cbr_ref[...] = c.astype(jnp.bfloat16).astype(jnp.float32)


def _prep(codebook):
    return pl.pallas_call(
        _prep_body,
        grid=(_K // _PREP_BLK,),
        in_specs=[pl.BlockSpec((_PREP_BLK, _D), lambda i: (i, 0))],
        out_specs=[
            pl.BlockSpec((_D, _PREP_BLK), lambda i: (0, i)),
            pl.BlockSpec((_PREP_BLK, _D), lambda i: (i, 0)),
        ],
        out_shape=[
            jax.ShapeDtypeStruct((_D, _K), jnp.bfloat16),
            jax.ShapeDtypeStruct((_K, _D), jnp.float32),
        ],
    )(codebook)


def _argmax_body(z_ref, cnt_ref, idx_ref):
    z = z_ref[...]
    n = jnp.sqrt(jnp.sum(z * z, axis=1, keepdims=True))
    zn = (z * pl.reciprocal(jnp.maximum(n, 1e-12), approx=_APPROX)
          ).astype(jnp.bfloat16)
    sims = lax.dot_general(
        zn, cnt_ref[...], (((1,), (0,)), ((), ())),
        preferred_element_type=jnp.float32,
        precision=lax.Precision.DEFAULT,
    )
    m = jnp.max(sims, axis=1, keepdims=True)
    iota = lax.broadcasted_iota(jnp.int32, sims.shape, 1)
    idx_ref[...] = jnp.min(jnp.where(sims == m, iota, _K), axis=1, keepdims=True)


def _argmax(z, cnt):
    return pl.pallas_call(
        _argmax_body,
        grid=(_N // _BT,),
        in_specs=[
            pl.BlockSpec((_BT, _D), lambda i: (i, 0)),
            pl.BlockSpec((_D, _K), lambda i: (0, 0)),
        ],
        out_specs=pl.BlockSpec((_BT, 1), lambda i: (i, 0)),
        out_shape=jax.ShapeDtypeStruct((_N, 1), jnp.int32),
    )(z, cnt)


def _gather_sc(table, idx):
    nw = 32
    bpw = _N // nw
    mesh = plsc.VectorSubcoreMesh(core_axis_name="c", subcore_axis_name="s")

    @functools.partial(
        pl.kernel, mesh=mesh,
        out_type=jax.ShapeDtypeStruct((_N, _D), jnp.float32),
        scratch_types=[
            pltpu.VMEM((bpw,), jnp.int32),
            pltpu.VMEM((bpw, _D), jnp.float32),
            pltpu.SemaphoreType.DMA,
        ],
    )
    def k(table_hbm, idx_hbm, out_hbm, idx_v, rows_v, sem):
        wid = lax.axis_index("s") * 2 + lax.axis_index("c")
        base = wid * bpw
        pltpu.sync_copy(idx_hbm.at[pl.ds(base, bpw)], idx_v)
        pltpu.async_copy(table_hbm.at[idx_v], rows_v, sem).wait()
        pltpu.sync_copy(rows_v, out_hbm.at[pl.ds(base, bpw)])

    return k(table, idx)


def _loss_body(z_ref, zq_ref, out_ref):
    i = pl.program_id(0)
    d = zq_ref[...] - z_ref[...]
    s = jnp.sum(d * d, axis=(0, 1), keepdims=True)

    @pl.when(i == 0)
    def _():
        out_ref[...] = s

    @pl.when(i > 0)
    def _():
        out_ref[...] += s


def _loss(z, zq):
    blk = 1024
    return pl.pallas_call(
        _loss_body,
        grid=(_N // blk,),
        in_specs=[
            pl.BlockSpec((blk, _D), lambda i: (i, 0)),
            pl.BlockSpec((blk, _D), lambda i: (i, 0)),
        ],
        out_specs=pl.BlockSpec((1, 1), lambda i: (0, 0)),
        out_shape=jax.ShapeDtypeStruct((1, 1), jnp.float32),
    )(z, zq)


def kernel(z, codebook):
    cnt, cbr = _prep(codebook)
    idx = _argmax(z, cnt).reshape(_N)
    zq = _gather_sc(cbr, idx)
    ssq = _loss(z, zq)
    loss = (_CW / (_N * _D)) * ssq[0, 0]
    return zq, loss
